# baseline (speedup 1.0000x reference)
import functools

import jax
import jax.numpy as jnp
from jax import lax
from jax.experimental import pallas as pl
from jax.experimental.pallas import tpu as pltpu
from jax.experimental.pallas import tpu_sc as plsc

L = 16
W = 16
NWORDS = W + 1
RANKR = 256
NCOL = 512
BIG = 1 << 20
MSIZE = NWORDS * RANKR


def _build_elim(num_cores, num_subcores):
    nworkers = num_cores * num_subcores
    assert 64 % nworkers == 0
    bpw = 64 // nworkers

    mesh = plsc.VectorSubcoreMesh(core_axis_name="c", subcore_axis_name="s")

    def body(words_hbm, rankt_hbm, so_hbm, ehat_hbm, M, Mrow, rankT, so_v, idxv, ehat_v):
        wid = lax.axis_index("s") * num_cores + lax.axis_index("c")
        iota = lax.iota(jnp.int32, L)
        zeros = jnp.zeros((L,), jnp.int32)
        bigv = jnp.full((L,), BIG, jnp.int32)
        row_stride = iota * RANKR
        iota_nw = iota * NWORDS

        for t in range(bpw):
            b = wid * bpw + t
            pltpu.sync_copy(words_hbm.at[pl.ds(b * MSIZE, MSIZE)], Mrow)
            pltpu.sync_copy(rankt_hbm.at[pl.ds(b * NCOL, NCOL)], rankT)
            pltpu.sync_copy(so_hbm.at[pl.ds(b * NCOL, NCOL)], so_v)

            for w in range(NWORDS):
                for tc in range(RANKR // L):
                    g = plsc.load_gather(
                        Mrow, [(tc * L) * NWORDS + iota_nw + w])
                    M[pl.ds(w * RANKR + tc * L, L)] = g

            def step(r, idx_vec):
                roww = plsc.load_gather(M, [row_stride + r])
                sw = plsc.load_gather(
                    M, [jnp.full((L,), W * RANKR + r, jnp.int32)])[0]

                acc = [bigv, bigv, bigv, bigv]
                iota32 = iota * 32
                for j in range(32):
                    hit = lax.shift_left(roww, 31 - j) < 0
                    rk = plsc.load_gather(rankT, [iota32 + j])
                    acc[j & 3] = jnp.minimum(
                        acc[j & 3], jnp.where(hit, rk, bigv))
                best = jnp.minimum(jnp.minimum(acc[0], acc[1]),
                                   jnp.minimum(acc[2], acc[3]))
                bmin = jnp.min(best)
                bmin = jnp.where(sw != 0, jnp.minimum(bmin, NCOL), bmin)

                piv = jnp.where(bmin >= BIG, 0, bmin).astype(jnp.int32)
                idx_vec = jnp.where(iota == (r & (L - 1)), piv, idx_vec)
                idxv[pl.ds((r >> 4) * L, L)] = idx_vec

                is_syn = bmin >= NCOL
                ci = jnp.where(is_syn, 0, bmin)
                col = plsc.load_gather(so_v, [jnp.full((L,), ci, jnp.int32)])[0]
                w_p = jnp.where(is_syn, W, lax.shift_right_logical(col, 5))
                sh31 = jnp.full((L,), jnp.where(is_syn, 31, 31 - (col & 31)))
                pbase = w_p * RANKR
                bws = [jnp.full((L,), roww[w]) for w in range(W)]
                bws.append(jnp.full((L,), sw))

                for tc in range(RANKR // L):
                    base = tc * L
                    negc = lax.shift_right_arithmetic(
                        lax.shift_left(M[pl.ds(pbase + base, L)], sh31), 31)
                    negc = jnp.where((base + iota) == r, 0, negc)
                    for w in range(NWORDS):
                        sl = M[pl.ds(w * RANKR + base, L)]
                        M[pl.ds(w * RANKR + base, L)] = sl ^ (bws[w] & negc)

                return idx_vec

            lax.fori_loop(0, RANKR, step, zeros)

            for tc in range(NCOL // L):
                ehat_v[pl.ds(tc * L, L)] = zeros
            for tc in range(RANKR // L):
                piv = idxv[pl.ds(tc * L, L)]
                valid = piv < NCOL
                cols = plsc.load_gather(so_v, [jnp.where(valid, piv, 0)])
                solw = M[pl.ds(W * RANKR + tc * L, L)] & 1
                plsc.store_scatter(ehat_v, [cols], solw, mask=valid)

            pltpu.sync_copy(ehat_v, ehat_hbm.at[pl.ds(b * NCOL, NCOL)])

    return pl.kernel(
        body,
        out_type=jax.ShapeDtypeStruct((64 * NCOL,), jnp.int32),
        mesh=mesh,
        compiler_params=pltpu.CompilerParams(
            use_tc_tiling_on_sc=False, needs_layout_passes=False),
        scratch_types=[
            pltpu.VMEM((MSIZE,), jnp.int32),
            pltpu.VMEM((MSIZE,), jnp.int32),
            pltpu.VMEM((NCOL,), jnp.int32),
            pltpu.VMEM((NCOL,), jnp.int32),
            pltpu.VMEM((RANKR,), jnp.int32),
            pltpu.VMEM((NCOL,), jnp.int32),
        ],
    )


def kernel(llr, pcm, s, bs):
    bs_static = llr.shape[0]
    sort_order = jnp.argsort(llr, axis=-1).astype(jnp.int32)
    inv_sort = jnp.argsort(sort_order, axis=-1).astype(jnp.int32)

    pcm = pcm.astype(jnp.int32)
    shifts = jnp.arange(32, dtype=jnp.int32)
    packed = jnp.sum(
        jnp.left_shift(pcm.reshape(bs_static, RANKR, W, 32), shifts),
        axis=-1, dtype=jnp.int32)
    syn = jnp.transpose(s, (1, 0)).astype(jnp.int32)
    words = jnp.concatenate([packed, syn[:, :, None]], axis=-1)

    info = plsc.get_sparse_core_info()
    elim = _build_elim(info.num_cores, info.num_subcores)
    ehat = elim(words.reshape(-1), inv_sort.reshape(-1), sort_order.reshape(-1))
    return ehat.reshape(bs_static, NCOL).astype(jnp.bool_)


# speedup vs baseline: 1.3629x; 1.3629x over previous
import functools

import jax
import jax.numpy as jnp
from jax import lax
from jax.experimental import pallas as pl
from jax.experimental.pallas import tpu as pltpu
from jax.experimental.pallas import tpu_sc as plsc

L = 16
W = 16
NWORDS = W + 1
RANKR = 256
NCOL = 512
BIG = 1 << 20
MSIZE = NWORDS * RANKR


def _build_elim(num_cores, num_subcores):
    nworkers = num_cores * num_subcores
    assert 64 % nworkers == 0
    bpw = 64 // nworkers

    mesh = plsc.VectorSubcoreMesh(core_axis_name="c", subcore_axis_name="s")

    def body(words_hbm, rankt_hbm, so_hbm, ehat_hbm, M, Mrow, rankT, so_v, idxv, ehat_v):
        wid = lax.axis_index("s") * num_cores + lax.axis_index("c")
        iota = lax.iota(jnp.int32, L)
        zeros = jnp.zeros((L,), jnp.int32)
        bigv = jnp.full((L,), BIG, jnp.int32)
        row_stride = iota * RANKR
        iota_nw = iota * NWORDS

        for t in range(bpw):
            b = wid * bpw + t
            pltpu.sync_copy(words_hbm.at[pl.ds(b * MSIZE, MSIZE)], Mrow)
            pltpu.sync_copy(rankt_hbm.at[pl.ds(b * NCOL, NCOL)], rankT)
            pltpu.sync_copy(so_hbm.at[pl.ds(b * NCOL, NCOL)], so_v)

            for w in range(NWORDS):
                for tc in range(RANKR // L):
                    g = plsc.load_gather(
                        Mrow, [(tc * L) * NWORDS + iota_nw + w])
                    M[pl.ds(w * RANKR + tc * L, L)] = g

            def step(r, idx_vec):
                roww = plsc.load_gather(M, [row_stride + r])
                sw = plsc.load_gather(
                    M, [jnp.full((L,), W * RANKR + r, jnp.int32)])[0]

                acc = [bigv, bigv, bigv, bigv]
                for j in range(32):
                    hit = lax.shift_left(roww, 31 - j) < 0
                    acc[j & 3] = jnp.minimum(
                        acc[j & 3],
                        jnp.where(hit, rankT[pl.ds(j * L, L)], bigv))
                best = jnp.minimum(jnp.minimum(acc[0], acc[1]),
                                   jnp.minimum(acc[2], acc[3]))
                bmin = jnp.min(best)
                bmin = jnp.where(sw != 0, jnp.minimum(bmin, NCOL), bmin)

                piv = jnp.where(bmin >= BIG, 0, bmin).astype(jnp.int32)
                idx_vec = jnp.where(iota == (r & (L - 1)), piv, idx_vec)
                idxv[pl.ds((r >> 4) * L, L)] = idx_vec

                is_syn = bmin >= NCOL
                ci = jnp.where(is_syn, 0, bmin)
                col = plsc.load_gather(so_v, [jnp.full((L,), ci, jnp.int32)])[0]
                w_p = jnp.where(is_syn, W, lax.shift_right_logical(col, 5))
                sh31 = jnp.full((L,), jnp.where(is_syn, 31, 31 - (col & 31)))
                pbase = w_p * RANKR
                bws = [jnp.full((L,), roww[w]) for w in range(W)]
                bws.append(jnp.full((L,), sw))

                for tc in range(RANKR // L):
                    base = tc * L
                    negc = lax.shift_right_arithmetic(
                        lax.shift_left(M[pl.ds(pbase + base, L)], sh31), 31)
                    negc = jnp.where((base + iota) == r, 0, negc)
                    for w in range(NWORDS):
                        sl = M[pl.ds(w * RANKR + base, L)]
                        M[pl.ds(w * RANKR + base, L)] = sl ^ (bws[w] & negc)

                return idx_vec

            lax.fori_loop(0, RANKR, step, zeros)

            for tc in range(NCOL // L):
                ehat_v[pl.ds(tc * L, L)] = zeros
            for tc in range(RANKR // L):
                piv = idxv[pl.ds(tc * L, L)]
                valid = piv < NCOL
                cols = plsc.load_gather(so_v, [jnp.where(valid, piv, 0)])
                solw = M[pl.ds(W * RANKR + tc * L, L)] & 1
                plsc.store_scatter(ehat_v, [cols], solw, mask=valid)

            pltpu.sync_copy(ehat_v, ehat_hbm.at[pl.ds(b * NCOL, NCOL)])

    return pl.kernel(
        body,
        out_type=jax.ShapeDtypeStruct((64 * NCOL,), jnp.int32),
        mesh=mesh,
        compiler_params=pltpu.CompilerParams(
            use_tc_tiling_on_sc=False, needs_layout_passes=False),
        scratch_types=[
            pltpu.VMEM((MSIZE,), jnp.int32),
            pltpu.VMEM((MSIZE,), jnp.int32),
            pltpu.VMEM((NCOL,), jnp.int32),
            pltpu.VMEM((NCOL,), jnp.int32),
            pltpu.VMEM((RANKR,), jnp.int32),
            pltpu.VMEM((NCOL,), jnp.int32),
        ],
    )


def kernel(llr, pcm, s, bs):
    bs_static = llr.shape[0]
    sort_order = jnp.argsort(llr, axis=-1).astype(jnp.int32)
    inv_sort = jnp.argsort(sort_order, axis=-1).astype(jnp.int32)

    pcm = pcm.astype(jnp.int32)
    shifts = jnp.arange(32, dtype=jnp.int32)
    packed = jnp.sum(
        jnp.left_shift(pcm.reshape(bs_static, RANKR, W, 32), shifts),
        axis=-1, dtype=jnp.int32)
    syn = jnp.transpose(s, (1, 0)).astype(jnp.int32)
    words = jnp.concatenate([packed, syn[:, :, None]], axis=-1)

    rankt = jnp.transpose(inv_sort.reshape(bs_static, W, 32), (0, 2, 1))

    info = plsc.get_sparse_core_info()
    elim = _build_elim(info.num_cores, info.num_subcores)
    ehat = elim(words.reshape(-1), rankt.reshape(-1), sort_order.reshape(-1))
    return ehat.reshape(bs_static, NCOL).astype(jnp.bool_)


# speedup vs baseline: 1.4773x; 1.0839x over previous
import functools

import jax
import jax.numpy as jnp
from jax import lax
from jax.experimental import pallas as pl
from jax.experimental.pallas import tpu as pltpu
from jax.experimental.pallas import tpu_sc as plsc

L = 16
W = 16
NWORDS = W + 1
RANKR = 256
NCOL = 512
BIG = 1 << 20
MSIZE = NWORDS * RANKR


def _build_elim(num_cores, num_subcores):
    nworkers = num_cores * num_subcores
    assert 64 % nworkers == 0
    bpw = 64 // nworkers

    mesh = plsc.VectorSubcoreMesh(core_axis_name="c", subcore_axis_name="s")

    def body(words_hbm, rankt_hbm, so_hbm, ehat_hbm, *scr):
        Ms = scr[0:bpw]
        rankTs = scr[bpw:2 * bpw]
        so_vs = scr[2 * bpw:3 * bpw]
        idxvs = scr[3 * bpw:4 * bpw]
        ehat_v = scr[4 * bpw]

        wid = lax.axis_index("s") * num_cores + lax.axis_index("c")
        iota = lax.iota(jnp.int32, L)
        zeros = jnp.zeros((L,), jnp.int32)
        bigv = jnp.full((L,), BIG, jnp.int32)
        row_stride = iota * RANKR

        for t in range(bpw):
            b = wid * bpw + t
            pltpu.sync_copy(words_hbm.at[pl.ds(b * MSIZE, MSIZE)], Ms[t])
            pltpu.sync_copy(rankt_hbm.at[pl.ds(b * NCOL, NCOL)], rankTs[t])
            pltpu.sync_copy(so_hbm.at[pl.ds(b * NCOL, NCOL)], so_vs[t])

        def one_step(M, rankT, so_v, idxv, r, idx_vec):
            roww = plsc.load_gather(M, [row_stride + r])
            sw = plsc.load_gather(
                M, [jnp.full((L,), W * RANKR + r, jnp.int32)])[0]

            acc = [bigv, bigv, bigv, bigv]
            for j in range(32):
                hit = lax.shift_left(roww, 31 - j) < 0
                acc[j & 3] = jnp.minimum(
                    acc[j & 3],
                    jnp.where(hit, rankT[pl.ds(j * L, L)], bigv))
            best = jnp.minimum(jnp.minimum(acc[0], acc[1]),
                               jnp.minimum(acc[2], acc[3]))
            bmin = jnp.min(best)
            bmin = jnp.where(sw != 0, jnp.minimum(bmin, NCOL), bmin)

            piv = jnp.where(bmin >= BIG, 0, bmin).astype(jnp.int32)
            idx_vec = jnp.where(iota == (r & (L - 1)), piv, idx_vec)
            idxv[pl.ds((r >> 4) * L, L)] = idx_vec

            is_syn = bmin >= NCOL
            ci = jnp.where(is_syn, 0, bmin)
            col = plsc.load_gather(so_v, [jnp.full((L,), ci, jnp.int32)])[0]
            w_p = jnp.where(is_syn, W, lax.shift_right_logical(col, 5))
            sh31 = jnp.full((L,), jnp.where(is_syn, 31, 31 - (col & 31)))
            pbase = w_p * RANKR
            bws = [jnp.full((L,), roww[w]) for w in range(W)]
            bws.append(jnp.full((L,), sw))

            for tc in range(RANKR // L):
                base = tc * L
                negc = lax.shift_right_arithmetic(
                    lax.shift_left(M[pl.ds(pbase + base, L)], sh31), 31)
                negc = jnp.where((base + iota) == r, 0, negc)
                for w in range(NWORDS):
                    sl = M[pl.ds(w * RANKR + base, L)]
                    M[pl.ds(w * RANKR + base, L)] = sl ^ (bws[w] & negc)

            return idx_vec

        def step(r, idx_vecs):
            return tuple(
                one_step(Ms[t], rankTs[t], so_vs[t], idxvs[t], r, idx_vecs[t])
                for t in range(bpw))

        lax.fori_loop(0, RANKR, step, (zeros,) * bpw)

        for t in range(bpw):
            b = wid * bpw + t
            for tc in range(NCOL // L):
                ehat_v[pl.ds(tc * L, L)] = zeros
            for tc in range(RANKR // L):
                piv = idxvs[t][pl.ds(tc * L, L)]
                valid = piv < NCOL
                cols = plsc.load_gather(so_vs[t], [jnp.where(valid, piv, 0)])
                solw = Ms[t][pl.ds(W * RANKR + tc * L, L)] & 1
                plsc.store_scatter(ehat_v, [cols], solw, mask=valid)
            pltpu.sync_copy(ehat_v, ehat_hbm.at[pl.ds(b * NCOL, NCOL)])

    scratch = (
        [pltpu.VMEM((MSIZE,), jnp.int32) for _ in range(bpw)] +
        [pltpu.VMEM((NCOL,), jnp.int32) for _ in range(bpw)] +
        [pltpu.VMEM((NCOL,), jnp.int32) for _ in range(bpw)] +
        [pltpu.VMEM((RANKR,), jnp.int32) for _ in range(bpw)] +
        [pltpu.VMEM((NCOL,), jnp.int32)]
    )
    return pl.kernel(
        body,
        out_type=jax.ShapeDtypeStruct((64 * NCOL,), jnp.int32),
        mesh=mesh,
        compiler_params=pltpu.CompilerParams(
            use_tc_tiling_on_sc=False, needs_layout_passes=False),
        scratch_types=scratch,
    )


def kernel(llr, pcm, s, bs):
    bs_static = llr.shape[0]
    sort_order = jnp.argsort(llr, axis=-1).astype(jnp.int32)
    inv_sort = jnp.argsort(sort_order, axis=-1).astype(jnp.int32)

    pcm = pcm.astype(jnp.int32)
    shifts = jnp.arange(32, dtype=jnp.int32)
    packed = jnp.sum(
        jnp.left_shift(pcm.reshape(bs_static, RANKR, W, 32), shifts),
        axis=-1, dtype=jnp.int32)
    syn = jnp.transpose(s, (1, 0)).astype(jnp.int32)
    words = jnp.concatenate([packed, syn[:, :, None]], axis=-1)
    words = jnp.transpose(words, (0, 2, 1))

    rankt = jnp.transpose(inv_sort.reshape(bs_static, W, 32), (0, 2, 1))

    info = plsc.get_sparse_core_info()
    elim = _build_elim(info.num_cores, info.num_subcores)
    ehat = elim(words.reshape(-1), rankt.reshape(-1), sort_order.reshape(-1))
    return ehat.reshape(bs_static, NCOL).astype(jnp.bool_)


# speedup vs baseline: 1.6173x; 1.0948x over previous
import functools

import jax
import jax.numpy as jnp
from jax import lax
from jax.experimental import pallas as pl
from jax.experimental.pallas import tpu as pltpu
from jax.experimental.pallas import tpu_sc as plsc

L = 16
W = 16
NWORDS = W + 1
RANKR = 256
NCOL = 512
BIG = 1 << 20
MSIZE = NWORDS * RANKR


def _build_elim(num_cores, num_subcores):
    nworkers = num_cores * num_subcores
    assert 64 % nworkers == 0
    bpw = 64 // nworkers

    mesh = plsc.VectorSubcoreMesh(core_axis_name="c", subcore_axis_name="s")

    def body(words_hbm, rankt_hbm, so_hbm, ehat_hbm, *scr):
        Ms = scr[0:bpw]
        rankTs = scr[bpw:2 * bpw]
        so_vs = scr[2 * bpw:3 * bpw]
        idxvs = scr[3 * bpw:4 * bpw]
        ehat_v = scr[4 * bpw]

        wid = lax.axis_index("s") * num_cores + lax.axis_index("c")
        iota = lax.iota(jnp.int32, L)
        zeros = jnp.zeros((L,), jnp.int32)
        bigv = jnp.full((L,), BIG, jnp.int32)
        row_stride = iota * RANKR

        for t in range(bpw):
            b = wid * bpw + t
            pltpu.sync_copy(words_hbm.at[pl.ds(b * MSIZE, MSIZE)], Ms[t])
            pltpu.sync_copy(rankt_hbm.at[pl.ds(b * NCOL, NCOL)], rankTs[t])
            pltpu.sync_copy(so_hbm.at[pl.ds(b * NCOL, NCOL)], so_vs[t])

        def one_step(M, rankT, so_v, idxv, r, idx_vec):
            roww = plsc.load_gather(M, [row_stride + r])
            sw = plsc.load_gather(
                M, [jnp.full((L,), W * RANKR + r, jnp.int32)])[0]

            acc = [bigv, bigv, bigv, bigv]
            for j in range(32):
                hit = lax.shift_left(roww, 31 - j) < 0
                acc[j & 3] = jnp.minimum(
                    acc[j & 3],
                    jnp.where(hit, rankT[pl.ds(j * L, L)], bigv))
            best = jnp.minimum(jnp.minimum(acc[0], acc[1]),
                               jnp.minimum(acc[2], acc[3]))
            bmin = jnp.min(best)
            bmin = jnp.where(sw != 0, jnp.minimum(bmin, NCOL), bmin)

            piv = jnp.where(bmin >= BIG, 0, bmin).astype(jnp.int32)
            idx_vec = jnp.where(iota == (r & (L - 1)), piv, idx_vec)
            idxv[pl.ds((r >> 4) * L, L)] = idx_vec

            is_syn = bmin >= NCOL
            ci = jnp.where(is_syn, 0, bmin)
            col = plsc.load_gather(so_v, [jnp.full((L,), ci, jnp.int32)])[0]
            w_p = jnp.where(is_syn, W, lax.shift_right_logical(col, 5))
            sh31 = jnp.full((L,), jnp.where(is_syn, 31, 31 - (col & 31)))
            pbase = w_p * RANKR
            bws = [jnp.full((L,), roww[w]) for w in range(W)]
            bws.append(jnp.full((L,), sw))

            def chunk(tc, _):
                base = tc * L
                negc = lax.shift_right_arithmetic(
                    lax.shift_left(M[pl.ds(pbase + base, L)], sh31), 31)
                negc = jnp.where((base + iota) > r, negc, 0)
                for w in range(NWORDS):
                    sl = M[pl.ds(w * RANKR + base, L)]
                    M[pl.ds(w * RANKR + base, L)] = sl ^ (bws[w] & negc)
                return 0

            lax.fori_loop(r >> 4, RANKR // L, chunk, 0)
            return idx_vec

        def step(r, idx_vecs):
            return tuple(
                one_step(Ms[t], rankTs[t], so_vs[t], idxvs[t], r, idx_vecs[t])
                for t in range(bpw))

        lax.fori_loop(0, RANKR, step, (zeros,) * bpw)

        def one_bstep(M, so_v, idxv, r):
            piv = plsc.load_gather(idxv, [jnp.full((L,), r, jnp.int32)])[0]
            synw = plsc.load_gather(
                M, [jnp.full((L,), W * RANKR + r, jnp.int32)])[0]
            is_syn = piv >= NCOL
            ci = jnp.where(is_syn, 0, piv)
            col = plsc.load_gather(so_v, [jnp.full((L,), ci, jnp.int32)])[0]
            w_p = jnp.where(is_syn, W, lax.shift_right_logical(col, 5))
            sh31 = jnp.full((L,), jnp.where(is_syn, 31, 31 - (col & 31)))
            pbase = w_p * RANKR

            @pl.when((synw & 1) != 0)
            def _():
                def bchunk(tc, _):
                    base = tc * L
                    cmask = lax.shift_right_arithmetic(
                        lax.shift_left(M[pl.ds(pbase + base, L)], sh31), 31)
                    upd = jnp.where((base + iota) < r, cmask, 0) & 1
                    s0 = M[pl.ds(W * RANKR + base, L)]
                    M[pl.ds(W * RANKR + base, L)] = s0 ^ upd
                    return 0

                lax.fori_loop(0, (r >> 4) + 1, bchunk, 0)

        def bstep(k, carry):
            r = RANKR - 1 - k
            for t in range(bpw):
                one_bstep(Ms[t], so_vs[t], idxvs[t], r)
            return carry

        lax.fori_loop(0, RANKR, bstep, 0)

        for t in range(bpw):
            b = wid * bpw + t
            for tc in range(NCOL // L):
                ehat_v[pl.ds(tc * L, L)] = zeros
            for tc in range(RANKR // L):
                piv = idxvs[t][pl.ds(tc * L, L)]
                valid = piv < NCOL
                cols = plsc.load_gather(so_vs[t], [jnp.where(valid, piv, 0)])
                solw = Ms[t][pl.ds(W * RANKR + tc * L, L)] & 1
                plsc.store_scatter(ehat_v, [cols], solw, mask=valid)
            pltpu.sync_copy(ehat_v, ehat_hbm.at[pl.ds(b * NCOL, NCOL)])

    scratch = (
        [pltpu.VMEM((MSIZE,), jnp.int32) for _ in range(bpw)] +
        [pltpu.VMEM((NCOL,), jnp.int32) for _ in range(bpw)] +
        [pltpu.VMEM((NCOL,), jnp.int32) for _ in range(bpw)] +
        [pltpu.VMEM((RANKR,), jnp.int32) for _ in range(bpw)] +
        [pltpu.VMEM((NCOL,), jnp.int32)]
    )
    return pl.kernel(
        body,
        out_type=jax.ShapeDtypeStruct((64 * NCOL,), jnp.int32),
        mesh=mesh,
        compiler_params=pltpu.CompilerParams(
            use_tc_tiling_on_sc=False, needs_layout_passes=False),
        scratch_types=scratch,
    )


def kernel(llr, pcm, s, bs):
    bs_static = llr.shape[0]
    sort_order = jnp.argsort(llr, axis=-1).astype(jnp.int32)
    inv_sort = jnp.argsort(sort_order, axis=-1).astype(jnp.int32)

    pcm = pcm.astype(jnp.int32)
    shifts = jnp.arange(32, dtype=jnp.int32)
    packed = jnp.sum(
        jnp.left_shift(pcm.reshape(bs_static, RANKR, W, 32), shifts),
        axis=-1, dtype=jnp.int32)
    syn = jnp.transpose(s, (1, 0)).astype(jnp.int32)
    words = jnp.concatenate([packed, syn[:, :, None]], axis=-1)
    words = jnp.transpose(words, (0, 2, 1))

    rankt = jnp.transpose(inv_sort.reshape(bs_static, W, 32), (0, 2, 1))

    info = plsc.get_sparse_core_info()
    elim = _build_elim(info.num_cores, info.num_subcores)
    ehat = elim(words.reshape(-1), rankt.reshape(-1), sort_order.reshape(-1))
    return ehat.reshape(bs_static, NCOL).astype(jnp.bool_)


# speedup vs baseline: 1.8428x; 1.1394x over previous
import functools

import jax
import jax.numpy as jnp
from jax import lax
from jax.experimental import pallas as pl
from jax.experimental.pallas import tpu as pltpu
from jax.experimental.pallas import tpu_sc as plsc

L = 16
W = 16
NWORDS = W + 1
RANKR = 256
NCOL = 512
BIG = 1 << 20
MSIZE = NWORDS * RANKR


def _build_elim(num_cores, num_subcores):
    nworkers = num_cores * num_subcores
    assert 64 % nworkers == 0
    bpw = 64 // nworkers

    mesh = plsc.VectorSubcoreMesh(core_axis_name="c", subcore_axis_name="s")

    def body(words_hbm, rankt_hbm, so_hbm, ehat_hbm, *scr):
        Ms = scr[0:bpw]
        rankTs = scr[bpw:2 * bpw]
        so_vs = scr[2 * bpw:3 * bpw]
        idxvs = scr[3 * bpw:4 * bpw]
        ehat_v = scr[4 * bpw]

        wid = lax.axis_index("s") * num_cores + lax.axis_index("c")
        iota = lax.iota(jnp.int32, L)
        zeros = jnp.zeros((L,), jnp.int32)
        bigv = jnp.full((L,), BIG, jnp.int32)
        row_stride = iota * RANKR

        for t in range(bpw):
            b = wid * bpw + t
            pltpu.sync_copy(words_hbm.at[pl.ds(b * MSIZE, MSIZE)], Ms[t])
            pltpu.sync_copy(rankt_hbm.at[pl.ds(b * NCOL, NCOL)], rankTs[t])
            pltpu.sync_copy(so_hbm.at[pl.ds(b * NCOL, NCOL)], so_vs[t])

        def one_step(M, rankT, so_v, idxv, r, idx_vec):
            roww = plsc.load_gather(M, [row_stride + r])
            sw = plsc.load_gather(
                M, [jnp.full((L,), W * RANKR + r, jnp.int32)])[0]

            acc = [bigv, bigv, bigv, bigv]
            for j in range(32):
                hit = lax.shift_left(roww, 31 - j) < 0
                acc[j & 3] = jnp.minimum(
                    acc[j & 3],
                    jnp.where(hit, rankT[pl.ds(j * L, L)], bigv))
            best = jnp.minimum(jnp.minimum(acc[0], acc[1]),
                               jnp.minimum(acc[2], acc[3]))
            bmin = jnp.min(best)
            bmin = jnp.where(sw != 0, jnp.minimum(bmin, NCOL), bmin)

            piv = jnp.where(bmin >= BIG, 0, bmin).astype(jnp.int32)
            idx_vec = jnp.where(iota == (r & (L - 1)), piv, idx_vec)
            idxv[pl.ds((r >> 4) * L, L)] = idx_vec

            is_syn = bmin >= NCOL
            ci = jnp.where(is_syn, 0, bmin)
            col = plsc.load_gather(so_v, [jnp.full((L,), ci, jnp.int32)])[0]
            w_p = jnp.where(is_syn, W, lax.shift_right_logical(col, 5))
            sh31 = jnp.full((L,), jnp.where(is_syn, 31, 31 - (col & 31)))
            pbase = w_p * RANKR
            bws = [jnp.full((L,), roww[w]) for w in range(W)]
            bws.append(jnp.full((L,), sw))

            @plsc.parallel_loop(r >> 4, RANKR // L)
            def chunk(tc):
                base = tc * L
                negc = lax.shift_right_arithmetic(
                    lax.shift_left(M[pl.ds(pbase + base, L)], sh31), 31)
                negc = jnp.where((base + iota) > r, negc, 0)
                for w in range(NWORDS):
                    sl = M[pl.ds(w * RANKR + base, L)]
                    M[pl.ds(w * RANKR + base, L)] = sl ^ (bws[w] & negc)

            return idx_vec

        def step(r, idx_vecs):
            return tuple(
                one_step(Ms[t], rankTs[t], so_vs[t], idxvs[t], r, idx_vecs[t])
                for t in range(bpw))

        lax.fori_loop(0, RANKR, step, (zeros,) * bpw)

        def one_bstep(M, so_v, idxv, r):
            piv = plsc.load_gather(idxv, [jnp.full((L,), r, jnp.int32)])[0]
            synw = plsc.load_gather(
                M, [jnp.full((L,), W * RANKR + r, jnp.int32)])[0]
            is_syn = piv >= NCOL
            ci = jnp.where(is_syn, 0, piv)
            col = plsc.load_gather(so_v, [jnp.full((L,), ci, jnp.int32)])[0]
            w_p = jnp.where(is_syn, W, lax.shift_right_logical(col, 5))
            sh31 = jnp.full((L,), jnp.where(is_syn, 31, 31 - (col & 31)))
            pbase = w_p * RANKR

            @pl.when((synw & 1) != 0)
            def _():
                @plsc.parallel_loop(0, (r >> 4) + 1)
                def bchunk(tc):
                    base = tc * L
                    cmask = lax.shift_right_arithmetic(
                        lax.shift_left(M[pl.ds(pbase + base, L)], sh31), 31)
                    upd = jnp.where((base + iota) < r, cmask, 0) & 1
                    s0 = M[pl.ds(W * RANKR + base, L)]
                    M[pl.ds(W * RANKR + base, L)] = s0 ^ upd

        def bstep(k, carry):
            r = RANKR - 1 - k
            for t in range(bpw):
                one_bstep(Ms[t], so_vs[t], idxvs[t], r)
            return carry

        lax.fori_loop(0, RANKR, bstep, 0)

        for t in range(bpw):
            b = wid * bpw + t
            for tc in range(NCOL // L):
                ehat_v[pl.ds(tc * L, L)] = zeros
            for tc in range(RANKR // L):
                piv = idxvs[t][pl.ds(tc * L, L)]
                valid = piv < NCOL
                cols = plsc.load_gather(so_vs[t], [jnp.where(valid, piv, 0)])
                solw = Ms[t][pl.ds(W * RANKR + tc * L, L)] & 1
                plsc.store_scatter(ehat_v, [cols], solw, mask=valid)
            pltpu.sync_copy(ehat_v, ehat_hbm.at[pl.ds(b * NCOL, NCOL)])

    scratch = (
        [pltpu.VMEM((MSIZE,), jnp.int32) for _ in range(bpw)] +
        [pltpu.VMEM((NCOL,), jnp.int32) for _ in range(bpw)] +
        [pltpu.VMEM((NCOL,), jnp.int32) for _ in range(bpw)] +
        [pltpu.VMEM((RANKR,), jnp.int32) for _ in range(bpw)] +
        [pltpu.VMEM((NCOL,), jnp.int32)]
    )
    return pl.kernel(
        body,
        out_type=jax.ShapeDtypeStruct((64 * NCOL,), jnp.int32),
        mesh=mesh,
        compiler_params=pltpu.CompilerParams(
            use_tc_tiling_on_sc=False, needs_layout_passes=False),
        scratch_types=scratch,
    )


def kernel(llr, pcm, s, bs):
    bs_static = llr.shape[0]
    sort_order = jnp.argsort(llr, axis=-1).astype(jnp.int32)
    inv_sort = jnp.argsort(sort_order, axis=-1).astype(jnp.int32)

    pcm = pcm.astype(jnp.int32)
    shifts = jnp.arange(32, dtype=jnp.int32)
    packed = jnp.sum(
        jnp.left_shift(pcm.reshape(bs_static, RANKR, W, 32), shifts),
        axis=-1, dtype=jnp.int32)
    syn = jnp.transpose(s, (1, 0)).astype(jnp.int32)
    words = jnp.concatenate([packed, syn[:, :, None]], axis=-1)
    words = jnp.transpose(words, (0, 2, 1))

    rankt = jnp.transpose(inv_sort.reshape(bs_static, W, 32), (0, 2, 1))

    info = plsc.get_sparse_core_info()
    elim = _build_elim(info.num_cores, info.num_subcores)
    ehat = elim(words.reshape(-1), rankt.reshape(-1), sort_order.reshape(-1))
    return ehat.reshape(bs_static, NCOL).astype(jnp.bool_)
